# Initial kernel scaffold; baseline (speedup 1.0000x reference)
import jax
import jax.numpy as jnp
from jax.experimental import pallas as pl


def kernel(s, v, pos, batch, edge_index_global, W_dscore, W_out):
    raise NotImplementedError("write your pallas kernel here")


# speedup vs baseline: 16.7931x; 16.7931x over previous
import functools

import jax
import jax.numpy as jnp
from jax import lax
from jax.experimental import pallas as pl
from jax.experimental.pallas import tpu as pltpu
from jax.experimental.pallas import tpu_sc as plsc

_NC = 2
_NS = 16
_NW = _NC * _NS
_L = 16


def _d_body(s_ref, w_ref, d_ref):
    x = s_ref[...]
    act = x * jax.nn.sigmoid(x)
    w = w_ref[...]
    d_ref[...] = jnp.sum(act * w, axis=1, keepdims=True)


def _compute_d(s, W_dscore):
    n, sdim = s.shape
    bn = 2000
    assert n % bn == 0
    return pl.pallas_call(
        _d_body,
        grid=(n // bn,),
        in_specs=[
            pl.BlockSpec((bn, sdim), lambda i: (i, 0)),
            pl.BlockSpec((1, sdim), lambda i: (0, 0)),
        ],
        out_specs=pl.BlockSpec((bn, 1), lambda i: (i, 0)),
        out_shape=jax.ShapeDtypeStruct((n, 1), jnp.float32),
    )(s, W_dscore)


def _edge_partials(tx, ty, tz, td, srcs, tgts, zeros1, npad):
    ep = srcs.shape[0]
    k = 2048
    assert ep % (_NW * k) == 0
    epw = ep // _NW
    nchunk = epw // k
    rows_per = npad // _NS

    mesh = plsc.VectorSubcoreMesh(core_axis_name="c", subcore_axis_name="s")

    @functools.partial(
        pl.kernel,
        out_type=jax.ShapeDtypeStruct((_NC, 4, npad), jnp.float32),
        mesh=mesh,
        compiler_params=pltpu.CompilerParams(use_tc_tiling_on_sc=False),
        scratch_types=[
            pltpu.VMEM((k,), jnp.int32),
            pltpu.VMEM((k,), jnp.int32),
            [pltpu.VMEM((k,), jnp.float32) for _ in range(4)],
            [pltpu.VMEM((k,), jnp.float32) for _ in range(4)],
            [pltpu.VMEM((k,), jnp.float32) for _ in range(3)],
            [pltpu.VMEM_SHARED((npad,), jnp.float32) for _ in range(4)],
            pltpu.SemaphoreType.DMA,
        ],
    )
    def edge_kernel(tx_hbm, ty_hbm, tz_hbm, td_hbm, src_hbm, tgt_hbm, zero_hbm,
                    out_hbm, idx_s, idx_t, g_s, g_t, o, acc, sem):
        cid = lax.axis_index("c")
        sid = lax.axis_index("s")
        wid = cid * _NS + sid
        sl = pl.ds(sid * rows_per, rows_per)

        for a in acc:
            pltpu.sync_copy(zero_hbm.at[sl], a.at[sl])
        plsc.subcore_barrier()

        base0 = wid * epw
        tabs = (tx_hbm, ty_hbm, tz_hbm, td_hbm)

        def chunk_body(j, _):
            b = base0 + j * k
            pltpu.sync_copy(src_hbm.at[pl.ds(b, k)], idx_s)
            pltpu.sync_copy(tgt_hbm.at[pl.ds(b, k)], idx_t)
            cps = [pltpu.async_copy(tab.at[idx], dst, sem)
                   for idx, bufs in ((idx_s, g_s), (idx_t, g_t))
                   for tab, dst in zip(tabs, bufs)]
            for cp in cps:
                cp.wait()

            def comp(i, _):
                s16 = pl.ds(i * _L, _L)
                w = g_s[3][s16] + g_t[3][s16]
                for c in range(3):
                    o[c][s16] = w * (g_s[c][s16] - g_t[c][s16])
                return 0

            lax.fori_loop(0, k // _L, comp, 0, unroll=4)
            for c in range(3):
                pltpu.sync_copy(o[c], acc[c].at[idx_t], add=True)
            return 0

        lax.fori_loop(0, nchunk, chunk_body, 0)
        plsc.subcore_barrier()

        for c, a in enumerate(acc):
            pltpu.sync_copy(a.at[sl], out_hbm.at[cid, c, sl])

    return edge_kernel(tx, ty, tz, td, srcs, tgts, zeros1)


def _fin_body(v2_ref, wbd_ref, p0_ref, p1_ref, o_ref):
    o_ref[...] = (
        jnp.dot(v2_ref[...], wbd_ref[...], preferred_element_type=jnp.float32)
        + p0_ref[...] + p1_ref[...]
    )


def _finalize(v2, wbd, p0, p1):
    n, k2 = v2.shape
    bn = 2000
    assert n % bn == 0
    return pl.pallas_call(
        _fin_body,
        grid=(n // bn,),
        in_specs=[
            pl.BlockSpec((bn, k2), lambda i: (i, 0)),
            pl.BlockSpec((k2, 4), lambda i: (0, 0)),
            pl.BlockSpec((bn, 4), lambda i: (i, 0)),
            pl.BlockSpec((bn, 4), lambda i: (i, 0)),
        ],
        out_specs=pl.BlockSpec((bn, 4), lambda i: (i, 0)),
        out_shape=jax.ShapeDtypeStruct((n, 4), jnp.float32),
    )(v2, wbd, p0, p1)


def kernel(s, v, pos, batch, edge_index_global, W_dscore, W_out):
    del batch
    n = s.shape[0]
    e = edge_index_global.shape[1]
    vdim = v.shape[2]

    d = _compute_d(s, W_dscore)
    tx, ty, tz = pos[:, 0], pos[:, 1], pos[:, 2]
    td = d[:, 0]

    k = 2048
    ep = ((e + _NW * k - 1) // (_NW * k)) * (_NW * k)
    srcs = jnp.concatenate(
        [edge_index_global[0], jnp.zeros((ep - e,), edge_index_global.dtype)])
    tgts = jnp.concatenate(
        [edge_index_global[1], jnp.zeros((ep - e,), edge_index_global.dtype)])

    npad = ((n + 128 * _NS - 1) // (128 * _NS)) * (128 * _NS)
    zeros1 = jnp.zeros((npad,), jnp.float32)
    part = _edge_partials(tx, ty, tz, td, srcs, tgts, zeros1, npad)

    v2 = v.reshape(n, 3 * vdim)
    wbd = jnp.zeros((3 * vdim, 4), jnp.float32)
    for i in range(3):
        wbd = wbd.at[i * vdim:(i + 1) * vdim, i].set(W_out[0])

    p0 = part[0, :, :n].T
    p1 = part[1, :, :n].T
    score4 = _finalize(v2, wbd, p0, p1)
    return score4[:, :3]
